# baseline (device time: 758303 ns/iter reference)
import functools

import jax
import jax.numpy as jnp
from jax import lax
from jax.experimental import pallas as pl
from jax.experimental.pallas import tpu as pltpu

N_DEV = 16
M_CHUNK = 512


def kernel(x, w_mat):
    m, k_shard = x.shape
    _, n = w_mat.shape

    def body(x_ref, w_ref, out_ref, comm_ref, send_sems, recv_sems):
        d = lax.axis_index("i")
        left = lax.rem(d - 1 + N_DEV, N_DEV)
        right = lax.rem(d + 1, N_DEV)

        barrier_sem = pltpu.get_barrier_semaphore()
        for nbr in (left, right):
            pl.semaphore_signal(
                barrier_sem, inc=1,
                device_id=(nbr,), device_id_type=pl.DeviceIdType.MESH,
            )
        pl.semaphore_wait(barrier_sem, 2)

        def partial_chunk(c):
            xs = x_ref[pl.ds(c * M_CHUNK, M_CHUNK), :]
            return jnp.dot(xs, w_ref[:, :], preferred_element_type=jnp.float32)

        for s in range(N_DEV - 1):
            send_slot = s % 2
            recv_slot = (s + 1) % 2
            c = lax.rem(d - 1 - s + 2 * N_DEV, N_DEV)
            part = partial_chunk(c)
            if s == 0:
                comm_ref[send_slot, :, :] = part.astype(jnp.bfloat16)
            else:
                acc = part + comm_ref[send_slot, :, :].astype(jnp.float32)
                comm_ref[send_slot, :, :] = acc.astype(jnp.bfloat16)
            rdma = pltpu.make_async_remote_copy(
                src_ref=comm_ref.at[send_slot],
                dst_ref=comm_ref.at[recv_slot],
                send_sem=send_sems.at[send_slot],
                recv_sem=recv_sems.at[recv_slot],
                device_id=(right,),
                device_id_type=pl.DeviceIdType.MESH,
            )
            rdma.start()
            rdma.wait()

        final = partial_chunk(d) + comm_ref[(N_DEV - 1) % 2, :, :].astype(jnp.float32)
        out_ref[:, :] = jnp.maximum(final, 0.0)

    return pl.pallas_call(
        body,
        out_shape=jax.ShapeDtypeStruct((M_CHUNK, n), jnp.float32),
        in_specs=[
            pl.BlockSpec(memory_space=pltpu.VMEM),
            pl.BlockSpec(memory_space=pltpu.VMEM),
        ],
        out_specs=pl.BlockSpec(memory_space=pltpu.VMEM),
        scratch_shapes=[
            pltpu.VMEM((2, M_CHUNK, n), jnp.bfloat16),
            pltpu.SemaphoreType.DMA((2,)),
            pltpu.SemaphoreType.DMA((2,)),
        ],
        compiler_params=pltpu.CompilerParams(collective_id=0),
    )(x, w_mat)


# device time: 459328 ns/iter; 1.6509x vs baseline; 1.6509x over previous
import jax
import jax.numpy as jnp
from jax import lax
from jax.experimental import pallas as pl
from jax.experimental.pallas import tpu as pltpu

N_DEV = 16
M_CHUNK = 512


def kernel(x, w_mat):
    x = x.astype(jnp.bfloat16)
    w_mat = w_mat.astype(jnp.bfloat16)
    m, k_shard = x.shape
    _, n = w_mat.shape
    hn = n // 2

    def body(x_ref, w_ref, out_ref, comm_r, comm_l,
             send_r, recv_r, send_l, recv_l):
        d = lax.axis_index("i")
        left = lax.rem(d - 1 + N_DEV, N_DEV)
        right = lax.rem(d + 1, N_DEV)

        barrier_sem = pltpu.get_barrier_semaphore()
        for nbr in (left, right):
            pl.semaphore_signal(
                barrier_sem, inc=1,
                device_id=(nbr,), device_id_type=pl.DeviceIdType.MESH,
            )
        pl.semaphore_wait(barrier_sem, 2)

        def parts_for_step(s):
            c_r = lax.rem(d - 1 - s + 2 * N_DEV, N_DEV)
            c_l = lax.rem(d + 1 + s, N_DEV)
            p_r = jnp.dot(x_ref[pl.ds(c_r * M_CHUNK, M_CHUNK), :],
                          w_ref[:, :hn], preferred_element_type=jnp.float32)
            p_l = jnp.dot(x_ref[pl.ds(c_l * M_CHUNK, M_CHUNK), :],
                          w_ref[:, hn:], preferred_element_type=jnp.float32)
            return p_r, p_l

        def final_parts():
            xs = x_ref[pl.ds(d * M_CHUNK, M_CHUNK), :]
            p_r = jnp.dot(xs, w_ref[:, :hn], preferred_element_type=jnp.float32)
            p_l = jnp.dot(xs, w_ref[:, hn:], preferred_element_type=jnp.float32)
            return p_r, p_l

        part_r, part_l = parts_for_step(0)
        for s in range(N_DEV - 1):
            ss = s % 2
            rs = (s + 1) % 2
            if s == 0:
                comm_r[ss, :, :] = part_r.astype(jnp.bfloat16)
                comm_l[ss, :, :] = part_l.astype(jnp.bfloat16)
            else:
                comm_r[ss, :, :] = (
                    part_r + comm_r[ss, :, :].astype(jnp.float32)
                ).astype(jnp.bfloat16)
                comm_l[ss, :, :] = (
                    part_l + comm_l[ss, :, :].astype(jnp.float32)
                ).astype(jnp.bfloat16)
            rdma_r = pltpu.make_async_remote_copy(
                src_ref=comm_r.at[ss], dst_ref=comm_r.at[rs],
                send_sem=send_r.at[ss], recv_sem=recv_r.at[rs],
                device_id=(right,), device_id_type=pl.DeviceIdType.MESH,
            )
            rdma_l = pltpu.make_async_remote_copy(
                src_ref=comm_l.at[ss], dst_ref=comm_l.at[rs],
                send_sem=send_l.at[ss], recv_sem=recv_l.at[rs],
                device_id=(left,), device_id_type=pl.DeviceIdType.MESH,
            )
            rdma_r.start()
            rdma_l.start()
            if s < N_DEV - 2:
                part_r, part_l = parts_for_step(s + 1)
            else:
                part_r, part_l = final_parts()
            rdma_r.wait()
            rdma_l.wait()

        last = (N_DEV - 1) % 2
        out_ref[:, :hn] = jnp.maximum(
            part_r + comm_r[last, :, :].astype(jnp.float32), 0.0)
        out_ref[:, hn:] = jnp.maximum(
            part_l + comm_l[last, :, :].astype(jnp.float32), 0.0)

    return pl.pallas_call(
        body,
        out_shape=jax.ShapeDtypeStruct((M_CHUNK, n), jnp.float32),
        in_specs=[
            pl.BlockSpec(memory_space=pltpu.VMEM),
            pl.BlockSpec(memory_space=pltpu.VMEM),
        ],
        out_specs=pl.BlockSpec(memory_space=pltpu.VMEM),
        scratch_shapes=[
            pltpu.VMEM((2, M_CHUNK, hn), jnp.bfloat16),
            pltpu.VMEM((2, M_CHUNK, hn), jnp.bfloat16),
            pltpu.SemaphoreType.DMA((2,)),
            pltpu.SemaphoreType.DMA((2,)),
            pltpu.SemaphoreType.DMA((2,)),
            pltpu.SemaphoreType.DMA((2,)),
        ],
        compiler_params=pltpu.CompilerParams(
            collective_id=0, vmem_limit_bytes=96 * 1024 * 1024
        ),
    )(x, w_mat)


# device time: 378001 ns/iter; 2.0061x vs baseline; 1.2152x over previous
import jax
import jax.numpy as jnp
from jax import lax
from jax.experimental import pallas as pl
from jax.experimental.pallas import tpu as pltpu

N_DEV = 16
M_CHUNK = 512
N_SUB = 2


def kernel(x, w_mat):
    x = x.astype(jnp.bfloat16)
    w_mat = w_mat.astype(jnp.bfloat16)
    m, k_shard = x.shape
    _, n = w_mat.shape
    hn = n // 2
    sw = hn // N_SUB

    def body(x_ref, w_ref, out_ref, comm_r, comm_l,
             send_r, recv_r, send_l, recv_l):
        d = lax.axis_index("i")
        left = lax.rem(d - 1 + N_DEV, N_DEV)
        right = lax.rem(d + 1, N_DEV)

        barrier_sem = pltpu.get_barrier_semaphore()
        for nbr in (left, right):
            pl.semaphore_signal(
                barrier_sem, inc=1,
                device_id=(nbr,), device_id_type=pl.DeviceIdType.MESH,
            )
        pl.semaphore_wait(barrier_sem, 2)

        def parts_for_step(s):
            c_r = lax.rem(d - 1 - s + 2 * N_DEV, N_DEV)
            c_l = lax.rem(d + 1 + s, N_DEV)
            p_r = jnp.dot(x_ref[pl.ds(c_r * M_CHUNK, M_CHUNK), :],
                          w_ref[:, :hn], preferred_element_type=jnp.float32)
            p_l = jnp.dot(x_ref[pl.ds(c_l * M_CHUNK, M_CHUNK), :],
                          w_ref[:, hn:], preferred_element_type=jnp.float32)
            return p_r, p_l

        def final_parts():
            xs = x_ref[pl.ds(d * M_CHUNK, M_CHUNK), :]
            p_r = jnp.dot(xs, w_ref[:, :hn], preferred_element_type=jnp.float32)
            p_l = jnp.dot(xs, w_ref[:, hn:], preferred_element_type=jnp.float32)
            return p_r, p_l

        def make_rdma(comm, send, recv, ss, sub, target):
            return pltpu.make_async_remote_copy(
                src_ref=comm.at[ss, :, pl.ds(sub * sw, sw)],
                dst_ref=comm.at[(ss + 1) % 2, :, pl.ds(sub * sw, sw)],
                send_sem=send.at[ss, sub], recv_sem=recv.at[(ss + 1) % 2, sub],
                device_id=(target,), device_id_type=pl.DeviceIdType.MESH,
            )

        part_r, part_l = parts_for_step(0)
        prev = {}
        for s in range(N_DEV - 1):
            ss = s % 2
            for sub in range(N_SUB):
                cols = pl.ds(sub * sw, sw)
                for key, comm, send, recv, part, tgt in (
                    ("r", comm_r, send_r, recv_r, part_r, right),
                    ("l", comm_l, send_l, recv_l, part_l, left),
                ):
                    if s == 0:
                        comm[ss, :, cols] = part[:, sub * sw:(sub + 1) * sw
                                                 ].astype(jnp.bfloat16)
                    else:
                        prev[key, sub].wait()
                        comm[ss, :, cols] = (
                            part[:, sub * sw:(sub + 1) * sw]
                            + comm[ss, :, cols].astype(jnp.float32)
                        ).astype(jnp.bfloat16)
                    rdma = make_rdma(comm, send, recv, ss, sub, tgt)
                    rdma.start()
                    prev[key, sub] = rdma
            if s < N_DEV - 2:
                part_r, part_l = parts_for_step(s + 1)
            else:
                part_r, part_l = final_parts()

        for sub in range(N_SUB):
            prev["r", sub].wait()
            prev["l", sub].wait()

        last = (N_DEV - 1) % 2
        out_ref[:, :hn] = jnp.maximum(
            part_r + comm_r[last, :, :].astype(jnp.float32), 0.0)
        out_ref[:, hn:] = jnp.maximum(
            part_l + comm_l[last, :, :].astype(jnp.float32), 0.0)

    return pl.pallas_call(
        body,
        out_shape=jax.ShapeDtypeStruct((M_CHUNK, n), jnp.float32),
        in_specs=[
            pl.BlockSpec(memory_space=pltpu.VMEM),
            pl.BlockSpec(memory_space=pltpu.VMEM),
        ],
        out_specs=pl.BlockSpec(memory_space=pltpu.VMEM),
        scratch_shapes=[
            pltpu.VMEM((2, M_CHUNK, hn), jnp.bfloat16),
            pltpu.VMEM((2, M_CHUNK, hn), jnp.bfloat16),
            pltpu.SemaphoreType.DMA((2, N_SUB)),
            pltpu.SemaphoreType.DMA((2, N_SUB)),
            pltpu.SemaphoreType.DMA((2, N_SUB)),
            pltpu.SemaphoreType.DMA((2, N_SUB)),
        ],
        compiler_params=pltpu.CompilerParams(
            collective_id=0, vmem_limit_bytes=96 * 1024 * 1024
        ),
    )(x, w_mat)


# device time: 375782 ns/iter; 2.0179x vs baseline; 1.0059x over previous
import jax
import jax.numpy as jnp
from jax import lax
from jax.experimental import pallas as pl
from jax.experimental.pallas import tpu as pltpu

N_DEV = 16
M_CHUNK = 512
N_SUB = 4


def kernel(x, w_mat):
    x = x.astype(jnp.bfloat16)
    w_mat = w_mat.astype(jnp.bfloat16)
    m, k_shard = x.shape
    _, n = w_mat.shape
    hn = n // 2
    sw = hn // N_SUB

    def body(x_ref, w_ref, out_ref, comm_r, comm_l,
             send_r, recv_r, send_l, recv_l):
        d = lax.axis_index("i")
        left = lax.rem(d - 1 + N_DEV, N_DEV)
        right = lax.rem(d + 1, N_DEV)

        barrier_sem = pltpu.get_barrier_semaphore()
        for nbr in (left, right):
            pl.semaphore_signal(
                barrier_sem, inc=1,
                device_id=(nbr,), device_id_type=pl.DeviceIdType.MESH,
            )
        pl.semaphore_wait(barrier_sem, 2)

        def parts_for_step(s):
            c_r = lax.rem(d - 1 - s + 2 * N_DEV, N_DEV)
            c_l = lax.rem(d + 1 + s, N_DEV)
            p_r = jnp.dot(x_ref[pl.ds(c_r * M_CHUNK, M_CHUNK), :],
                          w_ref[:, :hn], preferred_element_type=jnp.float32)
            p_l = jnp.dot(x_ref[pl.ds(c_l * M_CHUNK, M_CHUNK), :],
                          w_ref[:, hn:], preferred_element_type=jnp.float32)
            return p_r, p_l

        def final_parts():
            xs = x_ref[pl.ds(d * M_CHUNK, M_CHUNK), :]
            p_r = jnp.dot(xs, w_ref[:, :hn], preferred_element_type=jnp.float32)
            p_l = jnp.dot(xs, w_ref[:, hn:], preferred_element_type=jnp.float32)
            return p_r, p_l

        def make_rdma(comm, send, recv, ss, sub, target):
            return pltpu.make_async_remote_copy(
                src_ref=comm.at[ss, :, pl.ds(sub * sw, sw)],
                dst_ref=comm.at[(ss + 1) % 2, :, pl.ds(sub * sw, sw)],
                send_sem=send.at[ss, sub], recv_sem=recv.at[(ss + 1) % 2, sub],
                device_id=(target,), device_id_type=pl.DeviceIdType.MESH,
            )

        part_r, part_l = parts_for_step(0)
        prev = {}
        for s in range(N_DEV - 1):
            ss = s % 2
            for sub in range(N_SUB):
                cols = pl.ds(sub * sw, sw)
                for key, comm, send, recv, part, tgt in (
                    ("r", comm_r, send_r, recv_r, part_r, right),
                    ("l", comm_l, send_l, recv_l, part_l, left),
                ):
                    if s == 0:
                        comm[ss, :, cols] = part[:, sub * sw:(sub + 1) * sw
                                                 ].astype(jnp.bfloat16)
                    else:
                        prev[key, sub].wait()
                        comm[ss, :, cols] = (
                            part[:, sub * sw:(sub + 1) * sw]
                            + comm[ss, :, cols].astype(jnp.float32)
                        ).astype(jnp.bfloat16)
                    rdma = make_rdma(comm, send, recv, ss, sub, tgt)
                    rdma.start()
                    prev[key, sub] = rdma
            if s < N_DEV - 2:
                part_r, part_l = parts_for_step(s + 1)
            else:
                part_r, part_l = final_parts()

        last = (N_DEV - 1) % 2
        for sub in range(N_SUB):
            cols = pl.ds(sub * sw, sw)
            prev["r", sub].wait()
            out_ref[:, sub * sw:(sub + 1) * sw] = jnp.maximum(
                part_r[:, sub * sw:(sub + 1) * sw]
                + comm_r[last, :, cols].astype(jnp.float32), 0.0)
            prev["l", sub].wait()
            out_ref[:, hn + sub * sw:hn + (sub + 1) * sw] = jnp.maximum(
                part_l[:, sub * sw:(sub + 1) * sw]
                + comm_l[last, :, cols].astype(jnp.float32), 0.0)

    return pl.pallas_call(
        body,
        out_shape=jax.ShapeDtypeStruct((M_CHUNK, n), jnp.float32),
        in_specs=[
            pl.BlockSpec(memory_space=pltpu.VMEM),
            pl.BlockSpec(memory_space=pltpu.VMEM),
        ],
        out_specs=pl.BlockSpec(memory_space=pltpu.VMEM),
        scratch_shapes=[
            pltpu.VMEM((2, M_CHUNK, hn), jnp.bfloat16),
            pltpu.VMEM((2, M_CHUNK, hn), jnp.bfloat16),
            pltpu.SemaphoreType.DMA((2, N_SUB)),
            pltpu.SemaphoreType.DMA((2, N_SUB)),
            pltpu.SemaphoreType.DMA((2, N_SUB)),
            pltpu.SemaphoreType.DMA((2, N_SUB)),
        ],
        compiler_params=pltpu.CompilerParams(
            collective_id=0, vmem_limit_bytes=96 * 1024 * 1024
        ),
    )(x, w_mat)


# device time: 364955 ns/iter; 2.0778x vs baseline; 1.0297x over previous
import jax
import jax.numpy as jnp
from jax import lax
from jax.experimental import pallas as pl
from jax.experimental.pallas import tpu as pltpu

N_DEV = 16
M_CHUNK = 512
N_SUB = 4


def kernel(x, w_mat):
    m, k_shard = x.shape
    _, n = w_mat.shape
    hn = n // 2
    sw = hn // N_SUB

    def body(x_ref, w_ref, out_ref, comm_r, comm_l,
             send_r, recv_r, send_l, recv_l):
        d = lax.axis_index("i")
        left = lax.rem(d - 1 + N_DEV, N_DEV)
        right = lax.rem(d + 1, N_DEV)

        barrier_sem = pltpu.get_barrier_semaphore()
        for nbr in (left, right):
            pl.semaphore_signal(
                barrier_sem, inc=1,
                device_id=(nbr,), device_id_type=pl.DeviceIdType.MESH,
            )
        pl.semaphore_wait(barrier_sem, 2)

        def parts_for_step(s):
            c_r = lax.rem(d - 1 - s + 2 * N_DEV, N_DEV)
            c_l = lax.rem(d + 1 + s, N_DEV)
            p_r = jnp.dot(x_ref[pl.ds(c_r * M_CHUNK, M_CHUNK), :],
                          w_ref[:, :hn], preferred_element_type=jnp.float32)
            p_l = jnp.dot(x_ref[pl.ds(c_l * M_CHUNK, M_CHUNK), :],
                          w_ref[:, hn:], preferred_element_type=jnp.float32)
            return p_r, p_l

        def final_parts():
            xs = x_ref[pl.ds(d * M_CHUNK, M_CHUNK), :]
            p_r = jnp.dot(xs, w_ref[:, :hn], preferred_element_type=jnp.float32)
            p_l = jnp.dot(xs, w_ref[:, hn:], preferred_element_type=jnp.float32)
            return p_r, p_l

        def make_rdma(comm, send, recv, ss, sub, target):
            return pltpu.make_async_remote_copy(
                src_ref=comm.at[ss, :, pl.ds(sub * sw, sw)],
                dst_ref=comm.at[(ss + 1) % 2, :, pl.ds(sub * sw, sw)],
                send_sem=send.at[ss, sub], recv_sem=recv.at[(ss + 1) % 2, sub],
                device_id=(target,), device_id_type=pl.DeviceIdType.MESH,
            )

        part_r, part_l = parts_for_step(0)
        prev = {}
        for s in range(N_DEV - 1):
            ss = s % 2
            for sub in range(N_SUB):
                cols = pl.ds(sub * sw, sw)
                for key, comm, send, recv, part, tgt in (
                    ("r", comm_r, send_r, recv_r, part_r, right),
                    ("l", comm_l, send_l, recv_l, part_l, left),
                ):
                    if s == 0:
                        comm[ss, :, cols] = part[:, sub * sw:(sub + 1) * sw
                                                 ].astype(jnp.bfloat16)
                    else:
                        prev[key, sub].wait()
                        comm[ss, :, cols] = (
                            part[:, sub * sw:(sub + 1) * sw]
                            + comm[ss, :, cols].astype(jnp.float32)
                        ).astype(jnp.bfloat16)
                    rdma = make_rdma(comm, send, recv, ss, sub, tgt)
                    rdma.start()
                    prev[key, sub] = rdma
            if s < N_DEV - 2:
                part_r, part_l = parts_for_step(s + 1)
            else:
                part_r, part_l = final_parts()

        last = (N_DEV - 1) % 2
        for sub in range(N_SUB):
            cols = pl.ds(sub * sw, sw)
            prev["r", sub].wait()
            out_ref[:, sub * sw:(sub + 1) * sw] = jnp.maximum(
                part_r[:, sub * sw:(sub + 1) * sw]
                + comm_r[last, :, cols].astype(jnp.float32), 0.0)
            prev["l", sub].wait()
            out_ref[:, hn + sub * sw:hn + (sub + 1) * sw] = jnp.maximum(
                part_l[:, sub * sw:(sub + 1) * sw]
                + comm_l[last, :, cols].astype(jnp.float32), 0.0)

    return pl.pallas_call(
        body,
        out_shape=jax.ShapeDtypeStruct((M_CHUNK, n), jnp.float32),
        in_specs=[
            pl.BlockSpec(memory_space=pltpu.VMEM),
            pl.BlockSpec(memory_space=pltpu.VMEM),
        ],
        out_specs=pl.BlockSpec(memory_space=pltpu.VMEM),
        scratch_shapes=[
            pltpu.VMEM((2, M_CHUNK, hn), jnp.bfloat16),
            pltpu.VMEM((2, M_CHUNK, hn), jnp.bfloat16),
            pltpu.SemaphoreType.DMA((2, N_SUB)),
            pltpu.SemaphoreType.DMA((2, N_SUB)),
            pltpu.SemaphoreType.DMA((2, N_SUB)),
            pltpu.SemaphoreType.DMA((2, N_SUB)),
        ],
        compiler_params=pltpu.CompilerParams(
            collective_id=0, vmem_limit_bytes=96 * 1024 * 1024
        ),
    )(x, w_mat)


# device time: 363420 ns/iter; 2.0866x vs baseline; 1.0042x over previous
import jax
import jax.numpy as jnp
from jax import lax
from jax.experimental import pallas as pl
from jax.experimental.pallas import tpu as pltpu

N_DEV = 16
M_CHUNK = 512
N_SUB = 4


def kernel(x, w_mat):
    m, k_shard = x.shape
    _, n = w_mat.shape
    hn = n // 2
    sw = hn // N_SUB

    def body(x_ref, w_ref, out_ref, comm_r, comm_l,
             send_r, recv_r, send_l, recv_l):
        d = lax.axis_index("i")
        left = lax.rem(d - 1 + N_DEV, N_DEV)
        right = lax.rem(d + 1, N_DEV)

        barrier_sem = pltpu.get_barrier_semaphore()
        for nbr in (left, right):
            pl.semaphore_signal(
                barrier_sem, inc=1,
                device_id=(nbr,), device_id_type=pl.DeviceIdType.MESH,
            )
        pl.semaphore_wait(barrier_sem, 2)

        def parts_for_step(s):
            c_r = lax.rem(d - 1 - s + 2 * N_DEV, N_DEV)
            c_l = lax.rem(d + 1 + s, N_DEV)
            p_r = jnp.dot(x_ref[pl.ds(c_r * M_CHUNK, M_CHUNK), :],
                          w_ref[:, :hn], preferred_element_type=jnp.float32)
            p_l = jnp.dot(x_ref[pl.ds(c_l * M_CHUNK, M_CHUNK), :],
                          w_ref[:, hn:], preferred_element_type=jnp.float32)
            return p_r, p_l

        def final_parts():
            xs = x_ref[pl.ds(d * M_CHUNK, M_CHUNK), :]
            p_r = jnp.dot(xs, w_ref[:, :hn], preferred_element_type=jnp.float32)
            p_l = jnp.dot(xs, w_ref[:, hn:], preferred_element_type=jnp.float32)
            return p_r, p_l

        def make_rdma(comm, send, recv, ss, sub, target):
            return pltpu.make_async_remote_copy(
                src_ref=comm.at[ss, :, pl.ds(sub * sw, sw)],
                dst_ref=comm.at[(ss + 1) % 2, :, pl.ds(sub * sw, sw)],
                send_sem=send.at[ss, sub], recv_sem=recv.at[(ss + 1) % 2, sub],
                device_id=(target,), device_id_type=pl.DeviceIdType.MESH,
            )

        part_r = part_l = None
        prev = {}
        for s in range(N_DEV - 1):
            ss = s % 2
            for sub in range(N_SUB):
                cols = pl.ds(sub * sw, sw)
                for key, comm, send, recv, part, tgt in (
                    ("r", comm_r, send_r, recv_r, part_r, right),
                    ("l", comm_l, send_l, recv_l, part_l, left),
                ):
                    if s == 0:
                        c0 = (d - 1 if key == "r" else d + 1) + 2 * N_DEV
                        off = (0 if key == "r" else hn) + sub * sw
                        p = jnp.dot(
                            x_ref[pl.ds(lax.rem(c0, N_DEV) * M_CHUNK, M_CHUNK), :],
                            w_ref[:, off:off + sw],
                            preferred_element_type=jnp.float32)
                        comm[ss, :, cols] = p.astype(jnp.bfloat16)
                    else:
                        prev[key, sub].wait()
                        comm[ss, :, cols] = (
                            part[:, sub * sw:(sub + 1) * sw]
                            + comm[ss, :, cols].astype(jnp.float32)
                        ).astype(jnp.bfloat16)
                    rdma = make_rdma(comm, send, recv, ss, sub, tgt)
                    rdma.start()
                    prev[key, sub] = rdma
            if s < N_DEV - 2:
                part_r, part_l = parts_for_step(s + 1)
            else:
                part_r, part_l = final_parts()

        last = (N_DEV - 1) % 2
        for sub in range(N_SUB):
            cols = pl.ds(sub * sw, sw)
            prev["r", sub].wait()
            out_ref[:, sub * sw:(sub + 1) * sw] = jnp.maximum(
                part_r[:, sub * sw:(sub + 1) * sw]
                + comm_r[last, :, cols].astype(jnp.float32), 0.0)
            prev["l", sub].wait()
            out_ref[:, hn + sub * sw:hn + (sub + 1) * sw] = jnp.maximum(
                part_l[:, sub * sw:(sub + 1) * sw]
                + comm_l[last, :, cols].astype(jnp.float32), 0.0)

    return pl.pallas_call(
        body,
        out_shape=jax.ShapeDtypeStruct((M_CHUNK, n), jnp.float32),
        in_specs=[
            pl.BlockSpec(memory_space=pltpu.VMEM),
            pl.BlockSpec(memory_space=pltpu.VMEM),
        ],
        out_specs=pl.BlockSpec(memory_space=pltpu.VMEM),
        scratch_shapes=[
            pltpu.VMEM((2, M_CHUNK, hn), jnp.bfloat16),
            pltpu.VMEM((2, M_CHUNK, hn), jnp.bfloat16),
            pltpu.SemaphoreType.DMA((2, N_SUB)),
            pltpu.SemaphoreType.DMA((2, N_SUB)),
            pltpu.SemaphoreType.DMA((2, N_SUB)),
            pltpu.SemaphoreType.DMA((2, N_SUB)),
        ],
        compiler_params=pltpu.CompilerParams(
            collective_id=0, vmem_limit_bytes=96 * 1024 * 1024
        ),
    )(x, w_mat)
